# Initial kernel scaffold; baseline (speedup 1.0000x reference)
import jax
import jax.numpy as jnp
from jax.experimental import pallas as pl


def kernel(bbox_values, cu_seqlens, keep_ragged):
    raise NotImplementedError("write your pallas kernel here")


# speedup vs baseline: 2.7382x; 2.7382x over previous
import functools

import jax
import jax.numpy as jnp
from jax import lax
from jax.experimental import pallas as pl
from jax.experimental.pallas import tpu as pltpu
from jax.experimental.pallas import tpu_sc as plsc

B = 16
MAX_LEN = 4096
TOTAL = B * (MAX_LEN // 2)
FLAT = TOTAL * 4
ROW_F = MAX_LEN * 4
HALF_F = ROW_F // 2
IN_BUF = HALF_F + 16
PAD_IN = FLAT + IN_BUF + 16
NUM_CHUNKS = HALF_F // 16


def _body(flat_hbm, cu_hbm, out_hbm, cu_v, in_v, out_v, sem):
    cid = lax.axis_index("c")
    sid = lax.axis_index("s")
    b = sid
    half = cid

    pltpu.sync_copy(cu_hbm, cu_v)

    s = cu_v[pl.ds(b, 16)][0]
    e = cu_v[pl.ds(b + 1, 16)][0]

    o0 = half * HALF_F
    s0 = jnp.minimum(s * 4 + o0, FLAT)
    v = jnp.clip(e * 4 - s0, 0, HALF_F)
    a0 = (s0 // 8) * 8
    d = s0 - a0

    pltpu.sync_copy(flat_hbm.at[pl.ds(a0, IN_BUF)], in_v)

    lanes = lax.iota(jnp.int32, 16)

    def chunk(i, _):
        base = i * 16
        x = in_v[pl.ds(d + base, 16)]
        x = jnp.where(base + lanes < v, x, -1.0)
        out_v[pl.ds(base, 16)] = x
        return _

    lax.fori_loop(0, NUM_CHUNKS, chunk, None)

    pltpu.sync_copy(out_v, out_hbm.at[pl.ds(b * ROW_F + o0, HALF_F)])


@jax.jit
def _bbox_to_dense(flat_in, cu_pad):
    mesh = plsc.VectorSubcoreMesh(core_axis_name="c", subcore_axis_name="s")
    run = functools.partial(
        pl.kernel,
        out_type=jax.ShapeDtypeStruct((B * ROW_F,), jnp.float32),
        mesh=mesh,
        scratch_types=[
            pltpu.VMEM((32,), jnp.int32),
            pltpu.VMEM((IN_BUF,), jnp.float32),
            pltpu.VMEM((HALF_F,), jnp.float32),
            pltpu.SemaphoreType.DMA,
        ],
    )(_body)
    return run(flat_in, cu_pad)


def kernel(bbox_values, cu_seqlens, keep_ragged):
    flat_in = jnp.pad(bbox_values.reshape(-1), (0, PAD_IN - FLAT))
    cu_pad = jnp.pad(cu_seqlens.astype(jnp.int32), (0, 32 - (B + 1)))
    out = _bbox_to_dense(flat_in, cu_pad)
    return out.reshape(B, MAX_LEN, 4)


# speedup vs baseline: 2.7503x; 1.0044x over previous
import functools

import jax
import jax.numpy as jnp
from jax import lax
from jax.experimental import pallas as pl
from jax.experimental.pallas import tpu as pltpu
from jax.experimental.pallas import tpu_sc as plsc

B = 16
MAX_LEN = 4096
TOTAL = B * (MAX_LEN // 2)
FLAT = TOTAL * 4
ROW_F = MAX_LEN * 4
HALF_F = ROW_F // 2
IN_DMA = HALF_F + 16
IN_ALLOC = IN_DMA + 16
NUM_CHUNKS = HALF_F // 16


def _body(flat_hbm, cu_hbm, out_hbm, cu_v, in_v, out_v, sem):
    cid = lax.axis_index("c")
    sid = lax.axis_index("s")
    b = sid
    half = cid

    pltpu.sync_copy(cu_hbm, cu_v)

    s = cu_v[pl.ds(b, 16)][0]
    e = cu_v[pl.ds(b + 1, 16)][0]

    o0 = half * HALF_F
    s0 = jnp.minimum(s * 4 + o0, FLAT)
    v = jnp.clip(e * 4 - s0, 0, HALF_F)
    a0 = jnp.minimum((s0 // 8) * 8, FLAT - IN_DMA)
    d = s0 - a0

    pltpu.sync_copy(flat_hbm.at[pl.ds(a0, IN_DMA)], in_v.at[pl.ds(0, IN_DMA)])

    lanes = lax.iota(jnp.int32, 16)

    def chunk(i, _):
        base = i * 16
        off = jnp.minimum(d + base, IN_DMA)
        x = in_v[pl.ds(off, 16)]
        x = jnp.where(base + lanes < v, x, -1.0)
        out_v[pl.ds(base, 16)] = x
        return _

    lax.fori_loop(0, NUM_CHUNKS, chunk, None)

    pltpu.sync_copy(out_v, out_hbm.at[pl.ds(b * ROW_F + o0, HALF_F)])


@jax.jit
def _bbox_to_dense(flat_in, cu):
    mesh = plsc.VectorSubcoreMesh(core_axis_name="c", subcore_axis_name="s")
    run = functools.partial(
        pl.kernel,
        out_type=jax.ShapeDtypeStruct((B * ROW_F,), jnp.float32),
        mesh=mesh,
        scratch_types=[
            pltpu.VMEM((B + 1,), jnp.int32),
            pltpu.VMEM((IN_ALLOC,), jnp.float32),
            pltpu.VMEM((HALF_F,), jnp.float32),
            pltpu.SemaphoreType.DMA,
        ],
    )(_body)
    return run(flat_in, cu)


def kernel(bbox_values, cu_seqlens, keep_ragged):
    out = _bbox_to_dense(bbox_values.reshape(-1), cu_seqlens.astype(jnp.int32))
    return out.reshape(B, MAX_LEN, 4)
